# baseline (speedup 1.0000x reference)
import math
import numpy as np
import jax
import jax.numpy as jnp
from jax.experimental import pallas as pl
from jax.experimental.pallas import tpu as pltpu

_N = 512
_D = 128
_T_PAD = 256
_BI = 8


def _build_pe_np():
    pe = np.zeros((_N, _D), np.float32)
    position = np.arange(0, _N, dtype=np.float32)[:, None]
    div = np.exp(np.arange(0, _D, 2).astype(np.float32) * (-math.log(10000.0) / _D))
    pe[:, 0::2] = np.sin(position * div)
    pe[:, 1::2] = np.cos(position * div)
    return pe


_PE = jnp.asarray(_build_pe_np())


def _body(idx_ref, w_ref, pe_ref, out_ref, x_ref):
    @pl.when(pl.program_id(0) == 0)
    def _():
        idx = idx_ref[...]
        iota = jax.lax.broadcasted_iota(jnp.int32, (1, _T_PAD), 1)
        counts = (idx[:, 0:1] == iota).astype(jnp.float32)
        for k in range(1, 7):
            counts += (idx[:, k:k + 1] == iota).astype(jnp.float32)
        x_ref[...] = jnp.dot(counts, w_ref[...],
                             preferred_element_type=jnp.float32)
    out_ref[...] = x_ref[...][None, :, :] + pe_ref[...][:, None, :]


def kernel(time_features, hour_w, minute_w, second_w, day_w, month_w,
           year_w, weekday_w):
    offs = jnp.array([0, 24, 84, 144 - 1, 175 - 1, 187 - 2009, 190],
                     jnp.int32)
    cidx = time_features.astype(jnp.int32) + offs[None, :]
    cidx = jnp.pad(cidx, ((0, 0), (0, 1)), constant_values=_T_PAD - 1)
    w = jnp.concatenate([hour_w, minute_w, second_w, day_w, month_w,
                         year_w, weekday_w], axis=0)
    w = jnp.pad(w, ((0, _T_PAD - w.shape[0]), (0, 0)))

    return pl.pallas_call(
        _body,
        grid=(_N // _BI,),
        in_specs=[
            pl.BlockSpec((_N, 8), lambda i: (0, 0)),
            pl.BlockSpec((_T_PAD, _D), lambda i: (0, 0)),
            pl.BlockSpec((_BI, _D), lambda i: (i, 0)),
        ],
        out_specs=pl.BlockSpec((_BI, _N, _D), lambda i: (i, 0, 0)),
        out_shape=jax.ShapeDtypeStruct((_N, _N, _D), jnp.float32),
        scratch_shapes=[pltpu.VMEM((_N, _D), jnp.float32)],
    )(cidx, w, _PE)


# speedup vs baseline: 1.2024x; 1.2024x over previous
import math
import numpy as np
import jax
import jax.numpy as jnp
from jax.experimental import pallas as pl
from jax.experimental.pallas import tpu as pltpu

_N = 512
_D = 128
_T_PAD = 256
_BI = 32


def _build_pe_np():
    pe = np.zeros((_N, _D), np.float32)
    position = np.arange(0, _N, dtype=np.float32)[:, None]
    div = np.exp(np.arange(0, _D, 2).astype(np.float32) * (-math.log(10000.0) / _D))
    pe[:, 0::2] = np.sin(position * div)
    pe[:, 1::2] = np.cos(position * div)
    return pe


_PE = jnp.asarray(_build_pe_np())


def _body(idx_ref, w_ref, pe_ref, out_ref, x_ref):
    @pl.when(pl.program_id(0) == 0)
    def _():
        idx = idx_ref[...]
        iota = jax.lax.broadcasted_iota(jnp.int32, (1, _T_PAD), 1)
        counts = (idx[:, 0:1] == iota).astype(jnp.float32)
        for k in range(1, 7):
            counts += (idx[:, k:k + 1] == iota).astype(jnp.float32)
        x_ref[...] = jnp.dot(counts, w_ref[...],
                             preferred_element_type=jnp.float32)
    out_ref[...] = x_ref[...][None, :, :] + pe_ref[...][:, None, :]


def kernel(time_features, hour_w, minute_w, second_w, day_w, month_w,
           year_w, weekday_w):
    offs = jnp.array([0, 24, 84, 144 - 1, 175 - 1, 187 - 2009, 190],
                     jnp.int32)
    cidx = time_features.astype(jnp.int32) + offs[None, :]
    cidx = jnp.pad(cidx, ((0, 0), (0, 1)), constant_values=_T_PAD - 1)
    w = jnp.concatenate([hour_w, minute_w, second_w, day_w, month_w,
                         year_w, weekday_w], axis=0)
    w = jnp.pad(w, ((0, _T_PAD - w.shape[0]), (0, 0)))

    return pl.pallas_call(
        _body,
        grid=(_N // _BI,),
        in_specs=[
            pl.BlockSpec((_N, 8), lambda i: (0, 0)),
            pl.BlockSpec((_T_PAD, _D), lambda i: (0, 0)),
            pl.BlockSpec((_BI, _D), lambda i: (i, 0)),
        ],
        out_specs=pl.BlockSpec((_BI, _N, _D), lambda i: (i, 0, 0)),
        out_shape=jax.ShapeDtypeStruct((_N, _N, _D), jnp.float32),
        scratch_shapes=[pltpu.VMEM((_N, _D), jnp.float32)],
    )(cidx, w, _PE)


# speedup vs baseline: 1.2401x; 1.0313x over previous
import math
import numpy as np
import jax
import jax.numpy as jnp
from jax.experimental import pallas as pl
from jax.experimental.pallas import tpu as pltpu

_N = 512
_D = 128
_T_PAD = 256
_BI = 16


def _build_pe_np():
    pe = np.zeros((_N, _D), np.float32)
    position = np.arange(0, _N, dtype=np.float32)[:, None]
    div = np.exp(np.arange(0, _D, 2).astype(np.float32) * (-math.log(10000.0) / _D))
    pe[:, 0::2] = np.sin(position * div)
    pe[:, 1::2] = np.cos(position * div)
    return pe


_PE_NP = _build_pe_np()


def _body(idx_ref, w_ref, pe_ref, out_ref, x_ref):
    @pl.when(pl.program_id(0) == 0)
    def _():
        idx = idx_ref[...]
        iota = jax.lax.broadcasted_iota(jnp.int32, (1, _T_PAD), 1)
        counts = (idx[:, 0:1] == iota).astype(jnp.float32)
        for k in range(1, 7):
            counts += (idx[:, k:k + 1] == iota).astype(jnp.float32)
        x_ref[...] = jnp.dot(counts, w_ref[...],
                             preferred_element_type=jnp.float32)
    out_ref[...] = x_ref[...][None, :, :] + pe_ref[...][:, None, :]


def kernel(time_features, hour_w, minute_w, second_w, day_w, month_w,
           year_w, weekday_w):
    offs = jnp.array([0, 24, 84, 144 - 1, 175 - 1, 187 - 2009, 190],
                     jnp.int32)
    cidx = time_features.astype(jnp.int32) + offs[None, :]
    cidx = jnp.pad(cidx, ((0, 0), (0, 1)), constant_values=_T_PAD - 1)
    w = jnp.concatenate([hour_w, minute_w, second_w, day_w, month_w,
                         year_w, weekday_w], axis=0)
    w = jnp.pad(w, ((0, _T_PAD - w.shape[0]), (0, 0)))
    pe = jnp.asarray(_PE_NP)

    return pl.pallas_call(
        _body,
        grid=(_N // _BI,),
        in_specs=[
            pl.BlockSpec((_N, 8), lambda i: (0, 0)),
            pl.BlockSpec((_T_PAD, _D), lambda i: (0, 0)),
            pl.BlockSpec((_BI, _D), lambda i: (i, 0)),
        ],
        out_specs=pl.BlockSpec((_BI, _N, _D), lambda i: (i, 0, 0)),
        out_shape=jax.ShapeDtypeStruct((_N, _N, _D), jnp.float32),
        scratch_shapes=[pltpu.VMEM((_N, _D), jnp.float32)],
    )(cidx, w, pe)


# speedup vs baseline: 1.4288x; 1.1522x over previous
import math
import numpy as np
import jax
import jax.numpy as jnp
from jax.experimental import pallas as pl
from jax.experimental.pallas import tpu as pltpu

_N = 512
_D = 128
_T_PAD = 256
_BI = 16

_BANDS = (
    (0, 0),
    (32, 0),
    (96, 0),
    (160, 1),
    (192, 1),
    (208, 2009),
    (216, 0),
)


def _build_pe_np():
    pe = np.zeros((_N, _D), np.float32)
    position = np.arange(0, _N, dtype=np.float32)[:, None]
    div = np.exp(np.arange(0, _D, 2).astype(np.float32) * (-math.log(10000.0) / _D))
    pe[:, 0::2] = np.sin(position * div)
    pe[:, 1::2] = np.cos(position * div)
    return pe


_PE_NP = _build_pe_np()


def _body(tf_ref, hour_ref, minute_ref, second_ref, day_ref, month_ref,
          year_ref, weekday_ref, pe_ref, out_ref, x_ref, w_ref):
    @pl.when(pl.program_id(0) == 0)
    def _():
        w_ref[...] = jnp.zeros((_T_PAD, _D), jnp.float32)
        tables = (hour_ref, minute_ref, second_ref, day_ref, month_ref,
                  year_ref, weekday_ref)
        for (off, _), t in zip(_BANDS, tables):
            w_ref[pl.ds(off, t.shape[0]), :] = t[...]
        idx = tf_ref[...]
        iota = jax.lax.broadcasted_iota(jnp.int32, (1, _T_PAD), 1)
        counts = jnp.zeros((_N, _T_PAD), jnp.float32)
        for k, (off, adj) in enumerate(_BANDS):
            counts += (idx[:, k:k + 1] == iota - (off - adj)).astype(
                jnp.float32)
        x_ref[...] = jnp.dot(counts, w_ref[...],
                             preferred_element_type=jnp.float32)
    out_ref[...] = x_ref[...][None, :, :] + pe_ref[...][:, None, :]


def kernel(time_features, hour_w, minute_w, second_w, day_w, month_w,
           year_w, weekday_w):
    pe = jnp.asarray(_PE_NP)
    full = lambda shape: pl.BlockSpec(shape, lambda i: tuple(0 for _ in shape))
    return pl.pallas_call(
        _body,
        grid=(_N // _BI,),
        in_specs=[
            full((_N, 7)),
            full((24, _D)), full((60, _D)), full((60, _D)), full((31, _D)),
            full((12, _D)), full((3, _D)), full((7, _D)),
            pl.BlockSpec((_BI, _D), lambda i: (i, 0)),
        ],
        out_specs=pl.BlockSpec((_BI, _N, _D), lambda i: (i, 0, 0)),
        out_shape=jax.ShapeDtypeStruct((_N, _N, _D), jnp.float32),
        scratch_shapes=[pltpu.VMEM((_N, _D), jnp.float32),
                        pltpu.VMEM((_T_PAD, _D), jnp.float32)],
    )(time_features.astype(jnp.int32), hour_w, minute_w, second_w, day_w,
      month_w, year_w, weekday_w, pe)
